# baseline (speedup 1.0000x reference)
import functools

import jax
import jax.numpy as jnp
from jax import lax
from jax.experimental import pallas as pl
from jax.experimental.pallas import tpu as pltpu
from jax.experimental.pallas import tpu_sc as plsc

_LANE = 128


@functools.lru_cache(maxsize=None)
def _make_format(vocab, dim):
    info = plsc.get_sparse_core_info()
    nc, ns = info.num_cores, info.num_subcores
    nw = nc * ns
    vb = _LANE
    n_blocks = vocab // vb
    rem = vocab % vb
    per_w = (n_blocks // nw) & ~1
    extra = n_blocks - per_w * nw
    assert per_w >= 4 and extra + 1 <= nw and rem % 8 == 0

    mesh = plsc.VectorSubcoreMesh(core_axis_name="c", subcore_axis_name="s")

    @functools.partial(
        pl.kernel,
        mesh=mesh,
        out_type=jax.ShapeDtypeStruct((vocab // 8, 8, vb), jnp.float32),
        scratch_types=[
            pltpu.VMEM((dim // 8, 8, vb), jnp.float32),
            pltpu.VMEM((dim // 8, 8, vb), jnp.float32),
            pltpu.VMEM((vb // 8, 8, 2 * dim), jnp.float32),
            pltpu.VMEM((vb // 8, 8, 2 * dim), jnp.float32),
            pltpu.SemaphoreType.DMA,
            pltpu.SemaphoreType.DMA,
            pltpu.SemaphoreType.DMA,
            pltpu.SemaphoreType.DMA,
        ],
        compiler_params=pltpu.CompilerParams(
            use_tc_tiling_on_sc=True, needs_layout_passes=False
        ),
    )
    def format_kernel(wt_hbm, wlast_hbm, out_hbm, slab0, slab1, rowb0, rowb1,
                      gs0, gs1, ws0, ws1):
        wid = lax.axis_index("s") * nc + lax.axis_index("c")
        slabs = (slab0, slab1)
        rowbs = (rowb0, rowb1)
        gsems = (gs0, gs1)
        wsems = (ws0, ws1)
        iota = lax.iota(jnp.int32, 16)
        perms = tuple((iota + k) & 15 for k in range(16))
        tile_hi = iota // 8
        tile_lo = iota & 7

        def gfire(g, b):
            pltpu.async_copy(
                wt_hbm.at[:, :, pl.ds(g * vb, vb)], slabs[b], gsems[b]
            )

        def gwait(b):
            pltpu.make_async_copy(
                wt_hbm.at[:, :, pl.ds(0, vb)], slabs[b], gsems[b]
            ).wait()

        def wfire(g, b):
            pltpu.async_copy(
                rowbs[b], out_hbm.at[pl.ds(g * (vb // 8), vb // 8)], wsems[b]
            )

        def wwait(b):
            pltpu.make_async_copy(
                rowbs[b], out_hbm.at[pl.ds(0, vb // 8)], wsems[b]
            ).wait()

        def transpose(b):
            sv = slabs[b]
            rv = rowbs[b]

            def cbody(cb, carry):
                idx_c = iota + 16 * cb
                idx_t = tile_hi + 2 * cb
                for db in range(dim // 16):
                    for k in range(16):
                        d = perms[k] + 16 * db
                        vals = plsc.load_gather(
                            sv, [2 * db + perms[k] // 8, perms[k] & 7, idx_c]
                        )
                        plsc.store_scatter(rv, [idx_t, tile_lo, d], vals)
                return carry

            lax.fori_loop(0, vb // 16, cbody, 0)

        base = wid * per_w
        gfire(base, 0)
        gfire(base + 1, 1)

        for b in range(2):
            gwait(b)
            transpose(b)
            wfire(base + b, b)
            gfire(base + b + 2, b)

        def body(t2, carry):
            for b in range(2):
                t = 2 * t2 + b
                gwait(b)
                wwait(b)
                transpose(b)
                wfire(base + t, b)
                gfire(base + t + 2, b)
            return carry

        lax.fori_loop(1, per_w // 2 - 1, body, 0)

        for b in range(2):
            t = per_w - 2 + b
            gwait(b)
            wwait(b)
            transpose(b)
            wfire(base + t, b)
        wwait(0)
        wwait(1)

        @pl.when(wid < extra)
        def _():
            g = per_w * nw + wid
            pltpu.sync_copy(wt_hbm.at[:, :, pl.ds(g * vb, vb)], slab0)
            transpose(0)
            pltpu.sync_copy(
                rowb0, out_hbm.at[pl.ds(g * (vb // 8), vb // 8)]
            )

        if rem:
            @pl.when(wid == extra)
            def _():
                pltpu.sync_copy(wlast_hbm, rowb1.at[pl.ds(0, rem // 8)])
                pltpu.sync_copy(
                    rowb1.at[pl.ds(0, rem // 8)],
                    out_hbm.at[pl.ds((vocab - rem) // 8, rem // 8)],
                )

    return format_kernel


@functools.lru_cache(maxsize=None)
def _make_gather(seq, n_batch, vocab, dim):
    info = plsc.get_sparse_core_info()
    nc, ns = info.num_cores, info.num_subcores
    nw = nc * ns
    n_blk = n_batch // _LANE
    d_blk = dim // 8
    assert n_blk == nw and seq % 2 == 0

    mesh = plsc.VectorSubcoreMesh(core_axis_name="c", subcore_axis_name="s")

    @functools.partial(
        pl.kernel,
        mesh=mesh,
        out_type=jax.ShapeDtypeStruct((seq, d_blk, n_blk, 8, _LANE), jnp.float32),
        scratch_types=[
            pltpu.VMEM((seq, _LANE), jnp.int32),
            pltpu.VMEM((_LANE, 2 * dim), jnp.float32),
            pltpu.VMEM((_LANE, 2 * dim), jnp.float32),
            pltpu.VMEM((d_blk, 8, _LANE + 1), jnp.float32),
            pltpu.VMEM((d_blk, 8, _LANE + 1), jnp.float32),
            pltpu.SemaphoreType.DMA,
            pltpu.SemaphoreType.DMA,
            pltpu.SemaphoreType.DMA,
            pltpu.SemaphoreType.DMA,
        ],
        compiler_params=pltpu.CompilerParams(
            use_tc_tiling_on_sc=False, needs_layout_passes=False
        ),
    )
    def gather_kernel(ids_hbm, table_hbm, out_hbm, idx_v, rows0, rows1,
                      tile0, tile1, gs0, gs1, ws0, ws1):
        wid = lax.axis_index("s") * nc + lax.axis_index("c")
        pltpu.sync_copy(ids_hbm.at[:, pl.ds(wid * _LANE, _LANE)], idx_v)

        rows = (rows0, rows1)
        tiles = (tile0, tile1)
        gsems = (gs0, gs1)
        wsems = (ws0, ws1)
        iota = lax.iota(jnp.int32, 16)
        dblocks = tuple(
            (iota // 8 + (16 * db) // 8, iota % 8, 16 * db)
            for db in range(dim // 16)
        )

        def gfire(s, b):
            pltpu.async_copy(table_hbm.at[idx_v.at[s]], rows[b], gsems[b])

        def gwait(b):
            pltpu.make_async_copy(
                table_hbm.at[idx_v.at[0]], rows[b], gsems[b]
            ).wait()

        def wfire(s, b):
            pltpu.async_copy(
                tiles[b].at[:, :, pl.ds(0, _LANE)],
                out_hbm.at[s, :, wid],
                wsems[b],
            )

        def wwait(b):
            pltpu.make_async_copy(
                tiles[b].at[:, :, pl.ds(0, _LANE)],
                out_hbm.at[0, :, 0],
                wsems[b],
            ).wait()

        def transpose(b):
            rv = rows[b]
            tv = tiles[b]

            def cbody(c8, carry):
                c0 = 8 * c8
                for dc in range(8):
                    c = c0 + dc
                    cvec = jnp.zeros((16,), jnp.int32) + c
                    for i_idx, r_idx, d0 in dblocks:
                        vals = rv[c, pl.ds(d0, 16)]
                        plsc.store_scatter(tv, [i_idx, r_idx, cvec], vals)
                return carry

            lax.fori_loop(0, _LANE // 8, cbody, 0)

        gfire(0, 0)
        gfire(1, 1)

        for b in range(2):
            gwait(b)
            transpose(b)
            wfire(b, b)
            gfire(b + 2, b)

        def body(s2, carry):
            for b in range(2):
                s = 2 * s2 + b
                gwait(b)
                wwait(b)
                transpose(b)
                wfire(s, b)
                gfire(s + 2, b)
            return carry

        lax.fori_loop(1, seq // 2 - 1, body, 0)

        for b in range(2):
            s = seq - 2 + b
            gwait(b)
            wwait(b)
            transpose(b)
            wfire(s, b)
        wwait(0)
        wwait(1)

    return gather_kernel


def kernel(input_ids, weight):
    n_batch, seq = input_ids.shape
    vocab, dim = weight.shape
    ids_t = input_ids.T
    rem = vocab % _LANE
    wlast = (jnp.pad(weight[vocab - rem:], ((0, 0), (0, dim)))
             if rem else jnp.zeros((8, 2 * dim), jnp.float32))
    wt3 = weight.T.reshape(dim // 8, 8, vocab)
    wlast3 = wlast.reshape(-1, 8, 2 * dim)
    table = _make_format(vocab, dim)(wt3, wlast3).reshape(vocab, 2 * dim)
    out5d = _make_gather(seq, n_batch, vocab, dim)(ids_t, table)
    return out5d.transpose((2, 4, 0, 1, 3)).reshape(n_batch, seq, dim)


# speedup vs baseline: 1.0614x; 1.0614x over previous
import functools

import jax
import jax.numpy as jnp
from jax import lax
from jax.experimental import pallas as pl
from jax.experimental.pallas import tpu as pltpu
from jax.experimental.pallas import tpu_sc as plsc

_LANE = 128


@functools.lru_cache(maxsize=None)
def _make_gather(seq, n_batch, vocab, dim):
    info = plsc.get_sparse_core_info()
    nc, ns = info.num_cores, info.num_subcores
    nw = nc * ns
    n_blk = n_batch // _LANE
    d_blk = dim // 8
    assert n_blk == nw and seq % 2 == 0

    mesh = plsc.VectorSubcoreMesh(core_axis_name="c", subcore_axis_name="s")

    @functools.partial(
        pl.kernel,
        mesh=mesh,
        out_type=jax.ShapeDtypeStruct((seq, d_blk, n_blk, 8, _LANE), jnp.float32),
        scratch_types=[
            pltpu.VMEM((seq, _LANE), jnp.int32),
            pltpu.VMEM((2 * _LANE, dim), jnp.float32),
            pltpu.VMEM((2 * _LANE, dim), jnp.float32),
            pltpu.VMEM((2, d_blk, 8, _LANE + 1), jnp.float32),
            pltpu.VMEM((2, d_blk, 8, _LANE + 1), jnp.float32),
            pltpu.SemaphoreType.DMA,
            pltpu.SemaphoreType.DMA,
            pltpu.SemaphoreType.DMA,
            pltpu.SemaphoreType.DMA,
        ],
        compiler_params=pltpu.CompilerParams(
            use_tc_tiling_on_sc=False, needs_layout_passes=False
        ),
    )
    def gather_kernel(ids_hbm, table_hbm, out_hbm, idx_v, rows0, rows1,
                      tile0, tile1, gs0, gs1, ws0, ws1):
        wid = lax.axis_index("s") * nc + lax.axis_index("c")
        pltpu.sync_copy(ids_hbm.at[:, pl.ds(wid * _LANE, _LANE)], idx_v)

        rows = (rows0, rows1)
        tiles = (tile0, tile1)
        gsems = (gs0, gs1)
        wsems = (ws0, ws1)
        iota = lax.iota(jnp.int32, 16)
        dblocks = tuple(
            (iota // 8 + (16 * db) // 8, iota % 8, 16 * db)
            for db in range(dim // 16)
        )

        def gfire(ch, b):
            for cs in range(2):
                pltpu.async_copy(
                    table_hbm.at[idx_v.at[2 * ch + cs]],
                    rows[b].at[pl.ds(cs * _LANE, _LANE)],
                    gsems[b],
                )

        def gwait(b):
            for cs in range(2):
                pltpu.make_async_copy(
                    table_hbm.at[idx_v.at[0]],
                    rows[b].at[pl.ds(cs * _LANE, _LANE)],
                    gsems[b],
                ).wait()

        def wfire(ch, b):
            pltpu.async_copy(
                tiles[b].at[:, :, :, pl.ds(0, _LANE)],
                out_hbm.at[pl.ds(2 * ch, 2), :, wid],
                wsems[b],
            )

        def wwait(b):
            pltpu.make_async_copy(
                tiles[b].at[:, :, :, pl.ds(0, _LANE)],
                out_hbm.at[pl.ds(0, 2), :, 0],
                wsems[b],
            ).wait()

        def transpose(b):
            rv = rows[b]
            tv = tiles[b]

            def cbody(c8, carry):
                c0 = 8 * c8
                for cs in range(2):
                    cs_vec = jnp.zeros((16,), jnp.int32) + cs
                    for dc in range(8):
                        c = c0 + dc
                        cvec = jnp.zeros((16,), jnp.int32) + c
                        for i_idx, r_idx, d0 in dblocks:
                            vals = rv[cs * _LANE + c, pl.ds(d0, 16)]
                            plsc.store_scatter(
                                tv, [cs_vec, i_idx, r_idx, cvec], vals
                            )
                return carry

            lax.fori_loop(0, _LANE // 8, cbody, 0)

        n_ch = seq // 2
        gfire(0, 0)
        gfire(1, 1)

        for b in range(2):
            gwait(b)
            transpose(b)
            wfire(b, b)
            gfire(b + 2, b)

        def body(h, carry):
            for b in range(2):
                ch = 2 * h + b
                gwait(b)
                wwait(b)
                transpose(b)
                wfire(ch, b)
                gfire(ch + 2, b)
            return carry

        lax.fori_loop(1, n_ch // 2 - 1, body, 0)

        for b in range(2):
            ch = n_ch - 2 + b
            gwait(b)
            wwait(b)
            transpose(b)
            wfire(ch, b)
        wwait(0)
        wwait(1)

    return gather_kernel


def kernel(input_ids, weight):
    n_batch, seq = input_ids.shape
    vocab, dim = weight.shape
    ids_t = input_ids.T
    out5d = _make_gather(seq, n_batch, vocab, dim)(ids_t, weight)
    return out5d.transpose((2, 4, 0, 1, 3)).reshape(n_batch, seq, dim)


# speedup vs baseline: 1.1238x; 1.0588x over previous
import functools

import jax
import jax.numpy as jnp
from jax import lax
from jax.experimental import pallas as pl
from jax.experimental.pallas import tpu as pltpu
from jax.experimental.pallas import tpu_sc as plsc

_LANE = 128


@functools.lru_cache(maxsize=None)
def _make_gather(seq, n_batch, vocab, dim):
    info = plsc.get_sparse_core_info()
    nc, ns = info.num_cores, info.num_subcores
    nw = nc * ns
    n_blk = n_batch // _LANE
    d_blk = dim // 8
    assert n_blk == nw and seq % 2 == 0

    mesh = plsc.VectorSubcoreMesh(core_axis_name="c", subcore_axis_name="s")

    @functools.partial(
        pl.kernel,
        mesh=mesh,
        out_type=jax.ShapeDtypeStruct((seq, d_blk, n_blk, 8, _LANE), jnp.float32),
        scratch_types=[
            pltpu.VMEM((seq, _LANE), jnp.int32),
            pltpu.VMEM((_LANE, dim), jnp.float32),
            pltpu.VMEM((_LANE, dim), jnp.float32),
            pltpu.VMEM((d_blk, 8, _LANE + 1), jnp.float32),
            pltpu.VMEM((d_blk, 8, _LANE + 1), jnp.float32),
            pltpu.SemaphoreType.DMA,
            pltpu.SemaphoreType.DMA,
            pltpu.SemaphoreType.DMA,
            pltpu.SemaphoreType.DMA,
        ],
        compiler_params=pltpu.CompilerParams(
            use_tc_tiling_on_sc=False, needs_layout_passes=False
        ),
    )
    def gather_kernel(ids_hbm, table_hbm, out_hbm, idx_v, rows0, rows1,
                      tile0, tile1, gs0, gs1, ws0, ws1):
        wid = lax.axis_index("s") * nc + lax.axis_index("c")
        pltpu.sync_copy(ids_hbm.at[:, pl.ds(wid * _LANE, _LANE)], idx_v)

        rows = (rows0, rows1)
        tiles = (tile0, tile1)
        gsems = (gs0, gs1)
        wsems = (ws0, ws1)
        iota = lax.iota(jnp.int32, 16)
        dblocks = tuple(
            (iota // 8 + (16 * db) // 8, iota % 8, 16 * db)
            for db in range(dim // 16)
        )

        def gfire(s, b):
            pltpu.async_copy(table_hbm.at[idx_v.at[s]], rows[b], gsems[b])

        def gwait(b):
            pltpu.make_async_copy(
                table_hbm.at[idx_v.at[0]], rows[b], gsems[b]
            ).wait()

        def wfire(s, b):
            pltpu.async_copy(
                tiles[b].at[:, :, pl.ds(0, _LANE)],
                out_hbm.at[s, :, wid],
                wsems[b],
            )

        def wwait(b):
            pltpu.make_async_copy(
                tiles[b].at[:, :, pl.ds(0, _LANE)],
                out_hbm.at[0, :, 0],
                wsems[b],
            ).wait()

        def transpose(b):
            rv = rows[b]
            tv = tiles[b]

            def cbody(c8, carry):
                c0 = 8 * c8
                for dc in range(8):
                    c = c0 + dc
                    cvec = jnp.zeros((16,), jnp.int32) + c
                    for i_idx, r_idx, d0 in dblocks:
                        vals = rv[c, pl.ds(d0, 16)]
                        plsc.store_scatter(tv, [i_idx, r_idx, cvec], vals)
                return carry

            lax.fori_loop(0, _LANE // 8, cbody, 0)

        gfire(0, 0)
        gfire(1, 1)

        for b in range(2):
            gwait(b)
            transpose(b)
            wfire(b, b)
            gfire(b + 2, b)

        def body(s2, carry):
            for b in range(2):
                s = 2 * s2 + b
                gwait(b)
                wwait(b)
                transpose(b)
                wfire(s, b)
                gfire(s + 2, b)
            return carry

        lax.fori_loop(1, seq // 2 - 1, body, 0)

        for b in range(2):
            s = seq - 2 + b
            gwait(b)
            wwait(b)
            transpose(b)
            wfire(s, b)
        wwait(0)
        wwait(1)

    return gather_kernel


def kernel(input_ids, weight):
    n_batch, seq = input_ids.shape
    vocab, dim = weight.shape
    ids_t = input_ids.T
    out5d = _make_gather(seq, n_batch, vocab, dim)(ids_t, weight)
    return out5d.transpose((2, 4, 0, 1, 3)).reshape(n_batch, seq, dim)
